# baseline (speedup 1.0000x reference)
import jax
import jax.numpy as jnp
from jax import lax
from jax.experimental import pallas as pl
from jax.experimental.pallas import tpu as pltpu
from jax.experimental.pallas import tpu_sc as plsc

_NC = 2
_NS = 16
_NW = _NC * _NS
_BL = 128
_NBT = 78
_CHD = 128
_CHA = 64

_mesh = plsc.VectorSubcoreMesh(core_axis_name="c", subcore_axis_name="s")


def _stage(slab, row, off, dsb, nv):
    for k in range(nv):
        dsb[pl.ds(k * 16, 16)] = slab[row, pl.ds(off + k * 16, 16)]


def _deg_kernel(n, e):
    nxt = e // _BL - _NBT * _NW

    def body(ei_hbm, ones_hbm, z1_hbm, deg0, deg1, slab, xbuf, ones_v, stg,
             dsb, acc, sem):
        cid = lax.axis_index("c")
        sid = lax.axis_index("s")
        wid = cid * _NS + sid

        @pl.when(sid == 0)
        def _():
            pltpu.sync_copy(z1_hbm, stg)
            pltpu.sync_copy(stg, acc)

        pltpu.sync_copy(ones_hbm, ones_v)
        pltpu.sync_copy(ei_hbm.at[:, pl.ds(wid * _NBT * _BL, _NBT * _BL)],
                        slab)
        plsc.subcore_barrier()

        def chunk(j, carry):
            _stage(slab, 1, j * _CHD, dsb, _CHD // 16)
            pltpu.sync_copy(ones_v, acc.at[dsb], add=True)
            return carry

        lax.fori_loop(0, _NBT * _BL // _CHD, chunk, 0)

        @pl.when(wid < nxt)
        def _():
            pltpu.sync_copy(
                ei_hbm.at[:, pl.ds((_NBT * _NW + wid) * _BL, _BL)], xbuf)
            for q in range(_BL // _CHD):
                _stage(xbuf, 1, q * _CHD, dsb, _CHD // 16)
                pltpu.sync_copy(ones_v, acc.at[dsb], add=True)

        plsc.subcore_barrier()

        @pl.when(jnp.logical_and(sid == 0, cid == 0))
        def _():
            pltpu.sync_copy(acc, stg)
            pltpu.sync_copy(stg, deg0)

        @pl.when(jnp.logical_and(sid == 0, cid == 1))
        def _():
            pltpu.sync_copy(acc, stg)
            pltpu.sync_copy(stg, deg1)

    return pl.kernel(
        body,
        out_type=[jax.ShapeDtypeStruct((n,), jnp.float32),
                  jax.ShapeDtypeStruct((n,), jnp.float32)],
        mesh=_mesh,
        scratch_types=[
            pltpu.VMEM((2, _NBT * _BL), jnp.int32),
            pltpu.VMEM((2, _BL), jnp.int32),
            pltpu.VMEM((_CHD,), jnp.float32),
            pltpu.VMEM((n,), jnp.float32),
            pltpu.VMEM((_CHD,), jnp.int32),
            pltpu.VMEM_SHARED((n,), jnp.float32),
            pltpu.SemaphoreType.DMA,
        ],
    )


_WT = 10
_WC = 40


def _agg_kernel(n, e, d):
    cpt = _NBT * _BL // _CHA
    nxt = e // _BL - _NBT * _NW
    wr = n // _WT
    nwc = wr // _WC
    nv = _CHA // 16

    def body(ei_hbm, y_hbm, z2_hbm, out0, out1,
             slab, xbuf, buf0, buf1, ss0, ss1, ds0, ds1, zbuf, acc,
             sem0, sem1):
        cid = lax.axis_index("c")
        sid = lax.axis_index("s")
        wid = cid * _NS + sid

        @pl.when(sid < _WT)
        def _():
            pltpu.sync_copy(z2_hbm, zbuf)
            for k in range(nwc):
                pltpu.sync_copy(zbuf, acc.at[pl.ds(sid * wr + k * _WC, _WC)])

        pltpu.sync_copy(ei_hbm.at[:, pl.ds(wid * _NBT * _BL, _NBT * _BL)],
                        slab)
        plsc.subcore_barrier()

        def gather(ssb, buf, sem):
            pltpu.async_copy(y_hbm.at[ssb], buf, sem)

        def gwait(ssb, buf, sem):
            pltpu.make_async_copy(y_hbm.at[ssb], buf, sem).wait()

        _stage(slab, 0, 0, ss0, nv)
        gather(ss0, buf0, sem0)

        def chunk2(i, carry):
            j0 = 2 * i
            j1 = j0 + 1
            j2 = j0 + 2
            _stage(slab, 0, j1 * _CHA, ss1, nv)
            gather(ss1, buf1, sem1)
            _stage(slab, 1, j0 * _CHA, ds0, nv)
            gwait(ss0, buf0, sem0)
            pltpu.sync_copy(buf0, acc.at[ds0], add=True)

            @pl.when(j2 < cpt)
            def _():
                _stage(slab, 0, j2 * _CHA, ss0, nv)
                gather(ss0, buf0, sem0)

            _stage(slab, 1, j1 * _CHA, ds1, nv)
            gwait(ss1, buf1, sem1)
            pltpu.sync_copy(buf1, acc.at[ds1], add=True)
            return carry

        lax.fori_loop(0, cpt // 2, chunk2, 0)

        @pl.when(wid < nxt)
        def _():
            pltpu.sync_copy(
                ei_hbm.at[:, pl.ds((_NBT * _NW + wid) * _BL, _BL)], xbuf)
            for q in range(_BL // _CHA):
                _stage(xbuf, 0, q * _CHA, ss0, nv)
                pltpu.async_copy(y_hbm.at[ss0], buf0, sem0).wait()
                _stage(xbuf, 1, q * _CHA, ds0, nv)
                pltpu.sync_copy(buf0, acc.at[ds0], add=True)

        plsc.subcore_barrier()

        @pl.when(sid < _WT)
        def _():
            for k in range(nwc):
                r = sid * wr + k * _WC
                pltpu.sync_copy(acc.at[pl.ds(r, _WC)], zbuf)

                @pl.when(cid == 0)
                def _():
                    pltpu.sync_copy(zbuf, out0.at[pl.ds(r, _WC)])

                @pl.when(cid == 1)
                def _():
                    pltpu.sync_copy(zbuf, out1.at[pl.ds(r, _WC)])

    return pl.kernel(
        body,
        out_type=[jax.ShapeDtypeStruct((n, d), jnp.float32),
                  jax.ShapeDtypeStruct((n, d), jnp.float32)],
        mesh=_mesh,
        scratch_types=[
            pltpu.VMEM((2, _NBT * _BL), jnp.int32),
            pltpu.VMEM((2, _BL), jnp.int32),
            pltpu.VMEM((_CHA, d), jnp.float32),
            pltpu.VMEM((_CHA, d), jnp.float32),
            pltpu.VMEM((_CHA,), jnp.int32),
            pltpu.VMEM((_CHA,), jnp.int32),
            pltpu.VMEM((_CHA,), jnp.int32),
            pltpu.VMEM((_CHA,), jnp.int32),
            pltpu.VMEM((_WC, d), jnp.float32),
            pltpu.VMEM_SHARED((n, d), jnp.float32),
            pltpu.SemaphoreType.DMA,
            pltpu.SemaphoreType.DMA,
        ],
    )


def _xw_body(x_ref, w_ref, xw_ref):
    xw_ref[...] = jnp.dot(x_ref[...], w_ref[...],
                          preferred_element_type=jnp.float32)


def _dv_body(d0_ref, d1_ref, dv_ref):
    deg = d0_ref[...] + d1_ref[...] + 1.0
    dv_ref[...] = lax.rsqrt(deg)[:, None]


def _scale_body(xw_ref, dv_ref, y_ref):
    y_ref[...] = xw_ref[...] * dv_ref[...]


def _head_body(a0_ref, a1_ref, y_ref, dv_ref, bc_ref, wr_ref, br_ref, o_ref):
    s = a0_ref[...] + a1_ref[...] + y_ref[...]
    h = jnp.maximum(s * dv_ref[...] + bc_ref[...], 0.0)
    o_ref[...] = jnp.sum(h * wr_ref[...], axis=1, keepdims=True) + br_ref[...]


def kernel(x, edge_index, W_conv, b_conv, W_reg, b_reg):
    n, d = x.shape
    e = edge_index.shape[1]
    blk = n // 5

    ones_ch = jnp.ones((_CHD,), jnp.float32)
    z1 = jnp.zeros((n,), jnp.float32)
    z2 = jnp.zeros((_WC, d), jnp.float32)

    xw = pl.pallas_call(
        _xw_body,
        grid=(n // blk,),
        in_specs=[
            pl.BlockSpec((blk, d), lambda i: (i, 0)),
            pl.BlockSpec((d, d), lambda i: (0, 0)),
        ],
        out_specs=pl.BlockSpec((blk, d), lambda i: (i, 0)),
        out_shape=jax.ShapeDtypeStruct((n, d), jnp.float32),
    )(x, W_conv)

    deg0, deg1 = _deg_kernel(n, e)(edge_index, ones_ch, z1)

    dinv = pl.pallas_call(
        _dv_body,
        out_shape=jax.ShapeDtypeStruct((n, 1), jnp.float32),
    )(deg0, deg1)

    y = pl.pallas_call(
        _scale_body,
        grid=(n // blk,),
        in_specs=[
            pl.BlockSpec((blk, d), lambda i: (i, 0)),
            pl.BlockSpec((blk, 1), lambda i: (i, 0)),
        ],
        out_specs=pl.BlockSpec((blk, d), lambda i: (i, 0)),
        out_shape=jax.ShapeDtypeStruct((n, d), jnp.float32),
    )(xw, dinv)

    acc0, acc1 = _agg_kernel(n, e, d)(edge_index, y, z2)

    out = pl.pallas_call(
        _head_body,
        grid=(n // blk,),
        in_specs=[
            pl.BlockSpec((blk, d), lambda i: (i, 0)),
            pl.BlockSpec((blk, d), lambda i: (i, 0)),
            pl.BlockSpec((blk, d), lambda i: (i, 0)),
            pl.BlockSpec((blk, 1), lambda i: (i, 0)),
            pl.BlockSpec((1, d), lambda i: (0, 0)),
            pl.BlockSpec((1, d), lambda i: (0, 0)),
            pl.BlockSpec((1, 1), lambda i: (0, 0)),
        ],
        out_specs=pl.BlockSpec((blk, 1), lambda i: (i, 0)),
        out_shape=jax.ShapeDtypeStruct((n, 1), jnp.float32),
    )(acc0, acc1, y, dinv, b_conv.reshape(1, d), W_reg.reshape(1, d),
      b_reg.reshape(1, 1))

    return out


# speedup vs baseline: 1.1020x; 1.1020x over previous
import jax
import jax.numpy as jnp
from jax import lax
from jax.experimental import pallas as pl
from jax.experimental.pallas import tpu as pltpu
from jax.experimental.pallas import tpu_sc as plsc

_NC = 2
_NS = 16
_NW = _NC * _NS
_BL = 128
_NBT = 78
_CHD = 128
_CHA = 128
_NPH = 3

_mesh = plsc.VectorSubcoreMesh(core_axis_name="c", subcore_axis_name="s")


def _stage(slab, row, off, dsb, nv):
    for k in range(nv):
        dsb[pl.ds(k * 16, 16)] = slab[row, pl.ds(off + k * 16, 16)]


def _deg_kernel(n, e):
    nxt = e // _BL - _NBT * _NW

    def body(ei_hbm, ones_hbm, z1_hbm, deg0, deg1, slab, xbuf, ones_v, stg,
             dsb, acc, sem):
        cid = lax.axis_index("c")
        sid = lax.axis_index("s")
        wid = cid * _NS + sid

        @pl.when(sid == 0)
        def _():
            pltpu.sync_copy(z1_hbm, stg)
            pltpu.sync_copy(stg, acc)

        pltpu.sync_copy(ones_hbm, ones_v)
        pltpu.sync_copy(ei_hbm.at[:, pl.ds(wid * _NBT * _BL, _NBT * _BL)],
                        slab)
        plsc.subcore_barrier()

        def chunk(j, carry):
            _stage(slab, 1, j * _CHD, dsb, _CHD // 16)
            pltpu.sync_copy(ones_v, acc.at[dsb], add=True)
            return carry

        lax.fori_loop(0, _NBT * _BL // _CHD, chunk, 0)

        @pl.when(wid < nxt)
        def _():
            pltpu.sync_copy(
                ei_hbm.at[:, pl.ds((_NBT * _NW + wid) * _BL, _BL)], xbuf)
            for q in range(_BL // _CHD):
                _stage(xbuf, 1, q * _CHD, dsb, _CHD // 16)
                pltpu.sync_copy(ones_v, acc.at[dsb], add=True)

        plsc.subcore_barrier()

        @pl.when(jnp.logical_and(sid == 0, cid == 0))
        def _():
            pltpu.sync_copy(acc, stg)
            pltpu.sync_copy(stg, deg0)

        @pl.when(jnp.logical_and(sid == 0, cid == 1))
        def _():
            pltpu.sync_copy(acc, stg)
            pltpu.sync_copy(stg, deg1)

    return pl.kernel(
        body,
        out_type=[jax.ShapeDtypeStruct((n,), jnp.float32),
                  jax.ShapeDtypeStruct((n,), jnp.float32)],
        mesh=_mesh,
        scratch_types=[
            pltpu.VMEM((2, _NBT * _BL), jnp.int32),
            pltpu.VMEM((2, _BL), jnp.int32),
            pltpu.VMEM((_CHD,), jnp.float32),
            pltpu.VMEM((n,), jnp.float32),
            pltpu.VMEM((_CHD,), jnp.int32),
            pltpu.VMEM_SHARED((n,), jnp.float32),
            pltpu.SemaphoreType.DMA,
        ],
    )


_WT = 10
_WC = 40


def _agg_kernel(n, e, d):
    bps = _NBT // _NPH
    cps = bps * _BL // _CHA
    nxt = e // _BL - _NBT * _NW
    wr = n // _WT
    nwc = wr // _WC
    nv = _CHA // 16

    def body(ei_hbm, y_hbm, z2_hbm, out0, out1,
             slab, xbuf, buf0, buf1, ss0, ss1, ds0, ds1, zbuf, acc,
             sem0, sem1):
        cid = lax.axis_index("c")
        sid = lax.axis_index("s")
        wid = cid * _NS + sid

        @pl.when(sid < _WT)
        def _():
            pltpu.sync_copy(z2_hbm, zbuf)
            for k in range(nwc):
                pltpu.sync_copy(zbuf, acc.at[pl.ds(sid * wr + k * _WC, _WC)])

        plsc.subcore_barrier()

        def gather(ssb, buf, sem):
            pltpu.async_copy(y_hbm.at[ssb], buf, sem)

        def gwait(ssb, buf, sem):
            pltpu.make_async_copy(y_hbm.at[ssb], buf, sem).wait()

        for p in range(_NPH):
            pltpu.sync_copy(
                ei_hbm.at[:, pl.ds((wid * _NBT + p * bps) * _BL, bps * _BL)],
                slab)
            _stage(slab, 0, 0, ss0, nv)
            gather(ss0, buf0, sem0)

            def chunk2(i, carry):
                j0 = 2 * i
                j1 = j0 + 1
                j2 = j0 + 2
                _stage(slab, 0, j1 * _CHA, ss1, nv)
                gather(ss1, buf1, sem1)
                _stage(slab, 1, j0 * _CHA, ds0, nv)
                gwait(ss0, buf0, sem0)
                pltpu.sync_copy(buf0, acc.at[ds0], add=True)

                @pl.when(j2 < cps)
                def _():
                    _stage(slab, 0, j2 * _CHA, ss0, nv)
                    gather(ss0, buf0, sem0)

                _stage(slab, 1, j1 * _CHA, ds1, nv)
                gwait(ss1, buf1, sem1)
                pltpu.sync_copy(buf1, acc.at[ds1], add=True)
                return carry

            lax.fori_loop(0, cps // 2, chunk2, 0)

        @pl.when(wid < nxt)
        def _():
            pltpu.sync_copy(
                ei_hbm.at[:, pl.ds((_NBT * _NW + wid) * _BL, _BL)], xbuf)
            for q in range(_BL // _CHA):
                _stage(xbuf, 0, q * _CHA, ss0, nv)
                pltpu.async_copy(y_hbm.at[ss0], buf0, sem0).wait()
                _stage(xbuf, 1, q * _CHA, ds0, nv)
                pltpu.sync_copy(buf0, acc.at[ds0], add=True)

        plsc.subcore_barrier()

        @pl.when(sid < _WT)
        def _():
            for k in range(nwc):
                r = sid * wr + k * _WC
                pltpu.sync_copy(acc.at[pl.ds(r, _WC)], zbuf)

                @pl.when(cid == 0)
                def _():
                    pltpu.sync_copy(zbuf, out0.at[pl.ds(r, _WC)])

                @pl.when(cid == 1)
                def _():
                    pltpu.sync_copy(zbuf, out1.at[pl.ds(r, _WC)])

    return pl.kernel(
        body,
        out_type=[jax.ShapeDtypeStruct((n, d), jnp.float32),
                  jax.ShapeDtypeStruct((n, d), jnp.float32)],
        mesh=_mesh,
        scratch_types=[
            pltpu.VMEM((2, _NBT // _NPH * _BL), jnp.int32),
            pltpu.VMEM((2, _BL), jnp.int32),
            pltpu.VMEM((_CHA, d), jnp.float32),
            pltpu.VMEM((_CHA, d), jnp.float32),
            pltpu.VMEM((_CHA,), jnp.int32),
            pltpu.VMEM((_CHA,), jnp.int32),
            pltpu.VMEM((_CHA,), jnp.int32),
            pltpu.VMEM((_CHA,), jnp.int32),
            pltpu.VMEM((_WC, d), jnp.float32),
            pltpu.VMEM_SHARED((n, d), jnp.float32),
            pltpu.SemaphoreType.DMA,
            pltpu.SemaphoreType.DMA,
        ],
    )


def _xw_body(x_ref, w_ref, xw_ref):
    xw_ref[...] = jnp.dot(x_ref[...], w_ref[...],
                          preferred_element_type=jnp.float32)


def _dv_body(d0_ref, d1_ref, dv_ref):
    deg = d0_ref[...] + d1_ref[...] + 1.0
    dv_ref[...] = lax.rsqrt(deg)[:, None]


def _scale_body(xw_ref, dv_ref, y_ref):
    y_ref[...] = xw_ref[...] * dv_ref[...]


def _head_body(a0_ref, a1_ref, y_ref, dv_ref, bc_ref, wr_ref, br_ref, o_ref):
    s = a0_ref[...] + a1_ref[...] + y_ref[...]
    h = jnp.maximum(s * dv_ref[...] + bc_ref[...], 0.0)
    o_ref[...] = jnp.sum(h * wr_ref[...], axis=1, keepdims=True) + br_ref[...]


def kernel(x, edge_index, W_conv, b_conv, W_reg, b_reg):
    n, d = x.shape
    e = edge_index.shape[1]
    blk = n // 5

    ones_ch = jnp.ones((_CHD,), jnp.float32)
    z1 = jnp.zeros((n,), jnp.float32)
    z2 = jnp.zeros((_WC, d), jnp.float32)

    xw = pl.pallas_call(
        _xw_body,
        grid=(n // blk,),
        in_specs=[
            pl.BlockSpec((blk, d), lambda i: (i, 0)),
            pl.BlockSpec((d, d), lambda i: (0, 0)),
        ],
        out_specs=pl.BlockSpec((blk, d), lambda i: (i, 0)),
        out_shape=jax.ShapeDtypeStruct((n, d), jnp.float32),
    )(x, W_conv)

    deg0, deg1 = _deg_kernel(n, e)(edge_index, ones_ch, z1)

    dinv = pl.pallas_call(
        _dv_body,
        out_shape=jax.ShapeDtypeStruct((n, 1), jnp.float32),
    )(deg0, deg1)

    y = pl.pallas_call(
        _scale_body,
        grid=(n // blk,),
        in_specs=[
            pl.BlockSpec((blk, d), lambda i: (i, 0)),
            pl.BlockSpec((blk, 1), lambda i: (i, 0)),
        ],
        out_specs=pl.BlockSpec((blk, d), lambda i: (i, 0)),
        out_shape=jax.ShapeDtypeStruct((n, d), jnp.float32),
    )(xw, dinv)

    acc0, acc1 = _agg_kernel(n, e, d)(edge_index, y, z2)

    out = pl.pallas_call(
        _head_body,
        grid=(n // blk,),
        in_specs=[
            pl.BlockSpec((blk, d), lambda i: (i, 0)),
            pl.BlockSpec((blk, d), lambda i: (i, 0)),
            pl.BlockSpec((blk, d), lambda i: (i, 0)),
            pl.BlockSpec((blk, 1), lambda i: (i, 0)),
            pl.BlockSpec((1, d), lambda i: (0, 0)),
            pl.BlockSpec((1, d), lambda i: (0, 0)),
            pl.BlockSpec((1, 1), lambda i: (0, 0)),
        ],
        out_specs=pl.BlockSpec((blk, 1), lambda i: (i, 0)),
        out_shape=jax.ShapeDtypeStruct((n, 1), jnp.float32),
    )(acc0, acc1, y, dinv, b_conv.reshape(1, d), W_reg.reshape(1, d),
      b_reg.reshape(1, 1))

    return out


# speedup vs baseline: 1.1449x; 1.0389x over previous
import jax
import jax.numpy as jnp
from jax import lax
from jax.experimental import pallas as pl
from jax.experimental.pallas import tpu as pltpu
from jax.experimental.pallas import tpu_sc as plsc

_NC = 2
_NS = 16
_NW = _NC * _NS
_BL = 128
_NBT = 78
_CHD = 128
_DGR = 6
_CHA = 128
_NPH = 3

_mesh = plsc.VectorSubcoreMesh(core_axis_name="c", subcore_axis_name="s")


def _stage(slab, row, off, dsb, nv):
    for k in range(nv):
        dsb[pl.ds(k * 16, 16)] = slab[row, pl.ds(off + k * 16, 16)]


def _deg_kernel(n, e):
    nxt = e // _BL - _NBT * _NW

    def body(ei_hbm, ones_hbm, z1_hbm, deg0, deg1, slab, xbuf, ones_v, stg,
             dsb, acc, sem):
        cid = lax.axis_index("c")
        sid = lax.axis_index("s")
        wid = cid * _NS + sid

        @pl.when(sid == 0)
        def _():
            pltpu.sync_copy(z1_hbm, stg)
            pltpu.sync_copy(stg, acc)

        pltpu.sync_copy(ones_hbm, ones_v)
        pltpu.sync_copy(ei_hbm.at[:, pl.ds(wid * _NBT * _BL, _NBT * _BL)],
                        slab)
        plsc.subcore_barrier()

        def group(g, carry):
            for k in range(_DGR):
                _stage(slab, 1, (g * _DGR + k) * _CHD, dsb.at[k], _CHD // 16)
                pltpu.async_copy(ones_v, acc.at[dsb.at[k]], sem, add=True)
            for k in range(_DGR):
                pltpu.make_async_copy(ones_v, acc.at[dsb.at[k]], sem).wait()
            return carry

        lax.fori_loop(0, _NBT * _BL // _CHD // _DGR, group, 0)

        @pl.when(wid < nxt)
        def _():
            pltpu.sync_copy(
                ei_hbm.at[:, pl.ds((_NBT * _NW + wid) * _BL, _BL)], xbuf)
            for q in range(_BL // _CHD):
                _stage(xbuf, 1, q * _CHD, dsb.at[0], _CHD // 16)
                pltpu.sync_copy(ones_v, acc.at[dsb.at[0]], add=True)

        plsc.subcore_barrier()

        @pl.when(jnp.logical_and(sid == 0, cid == 0))
        def _():
            pltpu.sync_copy(acc, stg)
            pltpu.sync_copy(stg, deg0)

        @pl.when(jnp.logical_and(sid == 0, cid == 1))
        def _():
            pltpu.sync_copy(acc, stg)
            pltpu.sync_copy(stg, deg1)

    return pl.kernel(
        body,
        out_type=[jax.ShapeDtypeStruct((n,), jnp.float32),
                  jax.ShapeDtypeStruct((n,), jnp.float32)],
        mesh=_mesh,
        scratch_types=[
            pltpu.VMEM((2, _NBT * _BL), jnp.int32),
            pltpu.VMEM((2, _BL), jnp.int32),
            pltpu.VMEM((_CHD,), jnp.float32),
            pltpu.VMEM((n,), jnp.float32),
            pltpu.VMEM((_DGR, _CHD), jnp.int32),
            pltpu.VMEM_SHARED((n,), jnp.float32),
            pltpu.SemaphoreType.DMA,
        ],
    )


_WT = 10
_WC = 40


def _agg_kernel(n, e, d):
    bps = _NBT // _NPH
    cps = bps * _BL // _CHA
    nxt = e // _BL - _NBT * _NW
    wr = n // _WT
    nwc = wr // _WC
    nv = _CHA // 16

    def body(ei_hbm, y_hbm, z2_hbm, out0, out1,
             slab, xbuf, buf0, buf1, ss0, ss1, ds0, ds1, zbuf, acc,
             sem0, sem1):
        cid = lax.axis_index("c")
        sid = lax.axis_index("s")
        wid = cid * _NS + sid

        @pl.when(sid < _WT)
        def _():
            pltpu.sync_copy(z2_hbm, zbuf)
            for k in range(nwc):
                pltpu.sync_copy(zbuf, acc.at[pl.ds(sid * wr + k * _WC, _WC)])

        plsc.subcore_barrier()

        def gather(ssb, buf, sem):
            pltpu.async_copy(y_hbm.at[ssb], buf, sem)

        def gwait(ssb, buf, sem):
            pltpu.make_async_copy(y_hbm.at[ssb], buf, sem).wait()

        for p in range(_NPH):
            pltpu.sync_copy(
                ei_hbm.at[:, pl.ds((wid * _NBT + p * bps) * _BL, bps * _BL)],
                slab)
            _stage(slab, 0, 0, ss0, nv)
            gather(ss0, buf0, sem0)

            def chunk2(i, carry):
                j0 = 2 * i
                j1 = j0 + 1
                j2 = j0 + 2
                _stage(slab, 0, j1 * _CHA, ss1, nv)
                gather(ss1, buf1, sem1)
                _stage(slab, 1, j0 * _CHA, ds0, nv)
                gwait(ss0, buf0, sem0)
                pltpu.sync_copy(buf0, acc.at[ds0], add=True)

                @pl.when(j2 < cps)
                def _():
                    _stage(slab, 0, j2 * _CHA, ss0, nv)
                    gather(ss0, buf0, sem0)

                _stage(slab, 1, j1 * _CHA, ds1, nv)
                gwait(ss1, buf1, sem1)
                pltpu.sync_copy(buf1, acc.at[ds1], add=True)
                return carry

            lax.fori_loop(0, cps // 2, chunk2, 0)

        @pl.when(wid < nxt)
        def _():
            pltpu.sync_copy(
                ei_hbm.at[:, pl.ds((_NBT * _NW + wid) * _BL, _BL)], xbuf)
            for q in range(_BL // _CHA):
                _stage(xbuf, 0, q * _CHA, ss0, nv)
                pltpu.async_copy(y_hbm.at[ss0], buf0, sem0).wait()
                _stage(xbuf, 1, q * _CHA, ds0, nv)
                pltpu.sync_copy(buf0, acc.at[ds0], add=True)

        plsc.subcore_barrier()

        @pl.when(sid < _WT)
        def _():
            for k in range(nwc):
                r = sid * wr + k * _WC
                pltpu.sync_copy(acc.at[pl.ds(r, _WC)], zbuf)

                @pl.when(cid == 0)
                def _():
                    pltpu.sync_copy(zbuf, out0.at[pl.ds(r, _WC)])

                @pl.when(cid == 1)
                def _():
                    pltpu.sync_copy(zbuf, out1.at[pl.ds(r, _WC)])

    return pl.kernel(
        body,
        out_type=[jax.ShapeDtypeStruct((n, d), jnp.float32),
                  jax.ShapeDtypeStruct((n, d), jnp.float32)],
        mesh=_mesh,
        scratch_types=[
            pltpu.VMEM((2, _NBT // _NPH * _BL), jnp.int32),
            pltpu.VMEM((2, _BL), jnp.int32),
            pltpu.VMEM((_CHA, d), jnp.float32),
            pltpu.VMEM((_CHA, d), jnp.float32),
            pltpu.VMEM((_CHA,), jnp.int32),
            pltpu.VMEM((_CHA,), jnp.int32),
            pltpu.VMEM((_CHA,), jnp.int32),
            pltpu.VMEM((_CHA,), jnp.int32),
            pltpu.VMEM((_WC, d), jnp.float32),
            pltpu.VMEM_SHARED((n, d), jnp.float32),
            pltpu.SemaphoreType.DMA,
            pltpu.SemaphoreType.DMA,
        ],
    )


def _dv_body(d0_ref, d1_ref, dv_ref):
    deg = d0_ref[...] + d1_ref[...] + 1.0
    dv_ref[...] = lax.rsqrt(deg)[:, None]


def _y_body(x_ref, w_ref, dv_ref, y_ref):
    xw = jnp.dot(x_ref[...], w_ref[...], preferred_element_type=jnp.float32)
    y_ref[...] = xw * dv_ref[...]


def _head_body(a0_ref, a1_ref, y_ref, dv_ref, bc_ref, wr_ref, br_ref, o_ref):
    s = a0_ref[...] + a1_ref[...] + y_ref[...]
    h = jnp.maximum(s * dv_ref[...] + bc_ref[...], 0.0)
    o_ref[...] = jnp.sum(h * wr_ref[...], axis=1, keepdims=True) + br_ref[...]


def kernel(x, edge_index, W_conv, b_conv, W_reg, b_reg):
    n, d = x.shape
    e = edge_index.shape[1]
    blk = n // 5

    ones_ch = jnp.ones((_CHD,), jnp.float32)
    z1 = jnp.zeros((n,), jnp.float32)
    z2 = jnp.zeros((_WC, d), jnp.float32)

    deg0, deg1 = _deg_kernel(n, e)(edge_index, ones_ch, z1)

    dinv = pl.pallas_call(
        _dv_body,
        out_shape=jax.ShapeDtypeStruct((n, 1), jnp.float32),
    )(deg0, deg1)

    y = pl.pallas_call(
        _y_body,
        grid=(n // blk,),
        in_specs=[
            pl.BlockSpec((blk, d), lambda i: (i, 0)),
            pl.BlockSpec((d, d), lambda i: (0, 0)),
            pl.BlockSpec((blk, 1), lambda i: (i, 0)),
        ],
        out_specs=pl.BlockSpec((blk, d), lambda i: (i, 0)),
        out_shape=jax.ShapeDtypeStruct((n, d), jnp.float32),
    )(x, W_conv, dinv)

    acc0, acc1 = _agg_kernel(n, e, d)(edge_index, y, z2)

    out = pl.pallas_call(
        _head_body,
        grid=(n // blk,),
        in_specs=[
            pl.BlockSpec((blk, d), lambda i: (i, 0)),
            pl.BlockSpec((blk, d), lambda i: (i, 0)),
            pl.BlockSpec((blk, d), lambda i: (i, 0)),
            pl.BlockSpec((blk, 1), lambda i: (i, 0)),
            pl.BlockSpec((1, d), lambda i: (0, 0)),
            pl.BlockSpec((1, d), lambda i: (0, 0)),
            pl.BlockSpec((1, 1), lambda i: (0, 0)),
        ],
        out_specs=pl.BlockSpec((blk, 1), lambda i: (i, 0)),
        out_shape=jax.ShapeDtypeStruct((n, 1), jnp.float32),
    )(acc0, acc1, y, dinv, b_conv.reshape(1, d), W_reg.reshape(1, d),
      b_reg.reshape(1, 1))

    return out


# speedup vs baseline: 1.1787x; 1.0295x over previous
import jax
import jax.numpy as jnp
from jax import lax
from jax.experimental import pallas as pl
from jax.experimental.pallas import tpu as pltpu
from jax.experimental.pallas import tpu_sc as plsc

_NC = 2
_NS = 16
_NW = _NC * _NS
_BL = 128
_NBT = 78
_CHD = 128
_DGR = 6
_CHA = 128
_NPH = 3

_mesh = plsc.VectorSubcoreMesh(core_axis_name="c", subcore_axis_name="s")


def _stage(slab, row, off, dsb, nv):
    for k in range(nv):
        dsb[pl.ds(k * 16, 16)] = slab[row, pl.ds(off + k * 16, 16)]


def _deg_kernel(n, e):
    nxt = e // _BL - _NBT * _NW

    def body(ei_hbm, ones_hbm, z1_hbm, deg0, deg1, slab, xbuf, ones_v, stg,
             dsb, acc, sem):
        cid = lax.axis_index("c")
        sid = lax.axis_index("s")
        wid = cid * _NS + sid

        @pl.when(sid == 0)
        def _():
            pltpu.sync_copy(z1_hbm, stg)
            pltpu.sync_copy(stg, acc)

        pltpu.sync_copy(ones_hbm, ones_v)
        pltpu.sync_copy(ei_hbm.at[:, pl.ds(wid * _NBT * _BL, _NBT * _BL)],
                        slab)
        plsc.subcore_barrier()

        def group(g, carry):
            for k in range(_DGR):
                _stage(slab, 1, (g * _DGR + k) * _CHD, dsb.at[k], _CHD // 16)
                pltpu.async_copy(ones_v, acc.at[dsb.at[k]], sem, add=True)
            for k in range(_DGR):
                pltpu.make_async_copy(ones_v, acc.at[dsb.at[k]], sem).wait()
            return carry

        lax.fori_loop(0, _NBT * _BL // _CHD // _DGR, group, 0)

        @pl.when(wid < nxt)
        def _():
            pltpu.sync_copy(
                ei_hbm.at[:, pl.ds((_NBT * _NW + wid) * _BL, _BL)], xbuf)
            for q in range(_BL // _CHD):
                _stage(xbuf, 1, q * _CHD, dsb.at[0], _CHD // 16)
                pltpu.sync_copy(ones_v, acc.at[dsb.at[0]], add=True)

        plsc.subcore_barrier()

        @pl.when(jnp.logical_and(sid == 0, cid == 0))
        def _():
            pltpu.sync_copy(acc, stg)
            pltpu.sync_copy(stg, deg0)

        @pl.when(jnp.logical_and(sid == 0, cid == 1))
        def _():
            pltpu.sync_copy(acc, stg)
            pltpu.sync_copy(stg, deg1)

    return pl.kernel(
        body,
        out_type=[jax.ShapeDtypeStruct((n,), jnp.float32),
                  jax.ShapeDtypeStruct((n,), jnp.float32)],
        mesh=_mesh,
        scratch_types=[
            pltpu.VMEM((2, _NBT * _BL), jnp.int32),
            pltpu.VMEM((2, _BL), jnp.int32),
            pltpu.VMEM((_CHD,), jnp.float32),
            pltpu.VMEM((n,), jnp.float32),
            pltpu.VMEM((_DGR, _CHD), jnp.int32),
            pltpu.VMEM_SHARED((n,), jnp.float32),
            pltpu.SemaphoreType.DMA,
        ],
    )


_WT = 10
_WC = 40


def _agg_kernel(n, e, d):
    bps = _NBT // _NPH
    cps = bps * _BL // _CHA
    nxt = e // _BL - _NBT * _NW
    wr = n // _WT
    nwc = wr // _WC
    nv = _CHA // 16

    def body(ei_hbm, y_hbm, z2_hbm, out0, out1,
             slab, xbuf, buf0, buf1, ss0, ss1, ds0, ds1, zbuf, acc,
             sem0, sem1, wsem0, wsem1):
        cid = lax.axis_index("c")
        sid = lax.axis_index("s")
        wid = cid * _NS + sid

        @pl.when(sid < _WT)
        def _():
            pltpu.sync_copy(z2_hbm, zbuf)
            for k in range(nwc):
                pltpu.sync_copy(zbuf, acc.at[pl.ds(sid * wr + k * _WC, _WC)])

        pltpu.sync_copy(ei_hbm.at[:, pl.ds(wid * _NBT * _BL, bps * _BL)],
                        slab)
        plsc.subcore_barrier()

        def gather(ssb, buf, sem):
            pltpu.async_copy(y_hbm.at[ssb], buf, sem)

        def gwait(ssb, buf, sem):
            pltpu.make_async_copy(y_hbm.at[ssb], buf, sem).wait()

        for p in range(_NPH):
            if p > 0:
                pltpu.sync_copy(
                    ei_hbm.at[:, pl.ds((wid * _NBT + p * bps) * _BL,
                                       bps * _BL)],
                    slab)
            _stage(slab, 0, 0, ss0, nv)
            gather(ss0, buf0, sem0)

            def chunk2(i, carry):
                j0 = 2 * i
                j1 = j0 + 1
                j2 = j0 + 2
                _stage(slab, 0, j1 * _CHA, ss1, nv)
                gather(ss1, buf1, sem1)
                _stage(slab, 1, j0 * _CHA, ds0, nv)
                gwait(ss0, buf0, sem0)
                pltpu.sync_copy(buf0, acc.at[ds0], add=True)

                @pl.when(j2 < cps)
                def _():
                    _stage(slab, 0, j2 * _CHA, ss0, nv)
                    gather(ss0, buf0, sem0)

                _stage(slab, 1, j1 * _CHA, ds1, nv)
                gwait(ss1, buf1, sem1)
                pltpu.sync_copy(buf1, acc.at[ds1], add=True)
                return carry

            lax.fori_loop(0, cps // 2, chunk2, 0)

        @pl.when(wid < nxt)
        def _():
            pltpu.sync_copy(
                ei_hbm.at[:, pl.ds((_NBT * _NW + wid) * _BL, _BL)], xbuf)
            for q in range(_BL // _CHA):
                _stage(xbuf, 0, q * _CHA, ss0, nv)
                pltpu.async_copy(y_hbm.at[ss0], buf0, sem0).wait()
                _stage(xbuf, 1, q * _CHA, ds0, nv)
                pltpu.sync_copy(buf0, acc.at[ds0], add=True)

        plsc.subcore_barrier()

        def writeout(outref):
            bufs = [zbuf, buf0.at[pl.ds(0, _WC)]]
            rsem = [sem0, sem1]
            wsem = [wsem0, wsem1]

            def rslice(k):
                return acc.at[pl.ds(sid * wr + k * _WC, _WC)]

            def oslice(k):
                return outref.at[pl.ds(sid * wr + k * _WC, _WC)]

            pltpu.async_copy(rslice(0), bufs[0], rsem[0])
            for k in range(nwc):
                i = k % 2
                pltpu.make_async_copy(rslice(k), bufs[i], rsem[i]).wait()
                pltpu.async_copy(bufs[i], oslice(k), wsem[i])
                if k + 1 < nwc:
                    if k >= 1:
                        pltpu.make_async_copy(bufs[1 - i], oslice(k - 1),
                                              wsem[1 - i]).wait()
                    pltpu.async_copy(rslice(k + 1), bufs[1 - i], rsem[1 - i])
            for k in (nwc - 2, nwc - 1):
                pltpu.make_async_copy(bufs[k % 2], oslice(k),
                                      wsem[k % 2]).wait()

        @pl.when(jnp.logical_and(sid < _WT, cid == 0))
        def _():
            writeout(out0)

        @pl.when(jnp.logical_and(sid < _WT, cid == 1))
        def _():
            writeout(out1)

    return pl.kernel(
        body,
        out_type=[jax.ShapeDtypeStruct((n, d), jnp.float32),
                  jax.ShapeDtypeStruct((n, d), jnp.float32)],
        mesh=_mesh,
        scratch_types=[
            pltpu.VMEM((2, _NBT // _NPH * _BL), jnp.int32),
            pltpu.VMEM((2, _BL), jnp.int32),
            pltpu.VMEM((_CHA, d), jnp.float32),
            pltpu.VMEM((_CHA, d), jnp.float32),
            pltpu.VMEM((_CHA,), jnp.int32),
            pltpu.VMEM((_CHA,), jnp.int32),
            pltpu.VMEM((_CHA,), jnp.int32),
            pltpu.VMEM((_CHA,), jnp.int32),
            pltpu.VMEM((_WC, d), jnp.float32),
            pltpu.VMEM_SHARED((n, d), jnp.float32),
            pltpu.SemaphoreType.DMA,
            pltpu.SemaphoreType.DMA,
            pltpu.SemaphoreType.DMA,
            pltpu.SemaphoreType.DMA,
        ],
    )


def _dv_body(d0_ref, d1_ref, dv_ref):
    deg = d0_ref[...] + d1_ref[...] + 1.0
    dv_ref[...] = lax.rsqrt(deg)[:, None]


def _y_body(x_ref, w_ref, dv_ref, y_ref):
    xw = jnp.dot(x_ref[...], w_ref[...], preferred_element_type=jnp.float32)
    y_ref[...] = xw * dv_ref[...]


def _head_body(a0_ref, a1_ref, y_ref, dv_ref, bc_ref, wr_ref, br_ref, o_ref):
    s = a0_ref[...] + a1_ref[...] + y_ref[...]
    h = jnp.maximum(s * dv_ref[...] + bc_ref[...], 0.0)
    o_ref[...] = jnp.dot(h, wr_ref[...],
                         preferred_element_type=jnp.float32) + br_ref[...]


def kernel(x, edge_index, W_conv, b_conv, W_reg, b_reg):
    n, d = x.shape
    e = edge_index.shape[1]
    blk = n // 5

    ones_ch = jnp.ones((_CHD,), jnp.float32)
    z1 = jnp.zeros((n,), jnp.float32)
    z2 = jnp.zeros((_WC, d), jnp.float32)

    deg0, deg1 = _deg_kernel(n, e)(edge_index, ones_ch, z1)

    dinv = pl.pallas_call(
        _dv_body,
        out_shape=jax.ShapeDtypeStruct((n, 1), jnp.float32),
    )(deg0, deg1)

    y = pl.pallas_call(
        _y_body,
        grid=(n // blk,),
        in_specs=[
            pl.BlockSpec((blk, d), lambda i: (i, 0)),
            pl.BlockSpec((d, d), lambda i: (0, 0)),
            pl.BlockSpec((blk, 1), lambda i: (i, 0)),
        ],
        out_specs=pl.BlockSpec((blk, d), lambda i: (i, 0)),
        out_shape=jax.ShapeDtypeStruct((n, d), jnp.float32),
    )(x, W_conv, dinv)

    acc0, acc1 = _agg_kernel(n, e, d)(edge_index, y, z2)

    out = pl.pallas_call(
        _head_body,
        grid=(n // blk,),
        in_specs=[
            pl.BlockSpec((blk, d), lambda i: (i, 0)),
            pl.BlockSpec((blk, d), lambda i: (i, 0)),
            pl.BlockSpec((blk, d), lambda i: (i, 0)),
            pl.BlockSpec((blk, 1), lambda i: (i, 0)),
            pl.BlockSpec((d,), lambda i: (0,)),
            pl.BlockSpec((d, 1), lambda i: (0, 0)),
            pl.BlockSpec((1,), lambda i: (0,)),
        ],
        out_specs=pl.BlockSpec((blk, 1), lambda i: (i, 0)),
        out_shape=jax.ShapeDtypeStruct((n, 1), jnp.float32),
    )(acc0, acc1, y, dinv, b_conv, W_reg, b_reg)

    return out


# speedup vs baseline: 1.1972x; 1.0157x over previous
import jax
import jax.numpy as jnp
from jax import lax
from jax.experimental import pallas as pl
from jax.experimental.pallas import tpu as pltpu
from jax.experimental.pallas import tpu_sc as plsc

_NC = 2
_NS = 16
_NW = _NC * _NS
_BL = 128
_NBT = 78
_CHD = 128
_DGR = 13
_CHA = 128
_NPH = 3

_mesh = plsc.VectorSubcoreMesh(core_axis_name="c", subcore_axis_name="s")


def _stage(slab, row, off, dsb, nv):
    for k in range(nv):
        dsb[pl.ds(k * 16, 16)] = slab[row, pl.ds(off + k * 16, 16)]


def _deg_kernel(n, e):
    nxt = e // _BL - _NBT * _NW

    def body(ei_hbm, deg0, deg1, slab, xbuf, ones_v, stg, dsb, acc, sem):
        cid = lax.axis_index("c")
        sid = lax.axis_index("s")
        wid = cid * _NS + sid

        for k in range(_CHD // 16):
            ones_v[pl.ds(k * 16, 16)] = jnp.ones((16,), jnp.float32)

        @pl.when(sid == 0)
        def _():
            def zf(i, carry):
                stg[pl.ds(i * 16, 16)] = jnp.zeros((16,), jnp.float32)
                return carry

            lax.fori_loop(0, n // 16, zf, 0)
            pltpu.sync_copy(stg, acc)

        pltpu.sync_copy(ei_hbm.at[:, pl.ds(wid * _NBT * _BL, _NBT * _BL)],
                        slab)
        plsc.subcore_barrier()

        def group(g, carry):
            for k in range(_DGR):
                _stage(slab, 1, (g * _DGR + k) * _CHD, dsb.at[k], _CHD // 16)
                pltpu.async_copy(ones_v, acc.at[dsb.at[k]], sem, add=True)
            for k in range(_DGR):
                pltpu.make_async_copy(ones_v, acc.at[dsb.at[k]], sem).wait()
            return carry

        lax.fori_loop(0, _NBT * _BL // _CHD // _DGR, group, 0)

        @pl.when(wid < nxt)
        def _():
            pltpu.sync_copy(
                ei_hbm.at[:, pl.ds((_NBT * _NW + wid) * _BL, _BL)], xbuf)
            for q in range(_BL // _CHD):
                _stage(xbuf, 1, q * _CHD, dsb.at[0], _CHD // 16)
                pltpu.sync_copy(ones_v, acc.at[dsb.at[0]], add=True)

        plsc.subcore_barrier()

        @pl.when(jnp.logical_and(sid == 4, cid == 0))
        def _():
            pltpu.sync_copy(acc, stg)
            pltpu.sync_copy(stg, deg0)

        @pl.when(jnp.logical_and(sid == 4, cid == 1))
        def _():
            pltpu.sync_copy(acc, stg)
            pltpu.sync_copy(stg, deg1)

    return pl.kernel(
        body,
        out_type=[jax.ShapeDtypeStruct((n,), jnp.float32),
                  jax.ShapeDtypeStruct((n,), jnp.float32)],
        mesh=_mesh,
        scratch_types=[
            pltpu.VMEM((2, _NBT * _BL), jnp.int32),
            pltpu.VMEM((2, _BL), jnp.int32),
            pltpu.VMEM((_CHD,), jnp.float32),
            pltpu.VMEM((n,), jnp.float32),
            pltpu.VMEM((_DGR, _CHD), jnp.int32),
            pltpu.VMEM_SHARED((n,), jnp.float32),
            pltpu.SemaphoreType.DMA,
        ],
    )


_WT = 10
_WC = 40


def _agg_kernel(n, e, d):
    bps = _NBT // _NPH
    cps = bps * _BL // _CHA
    nxt = e // _BL - _NBT * _NW
    wr = n // _WT
    nwc = wr // _WC
    nv = _CHA // 16

    zr = n // _NS

    def body(ei_hbm, y_hbm, out0, out1,
             slab, xbuf, buf0, buf1, ss0, ss1, ds0, ds1, zbuf, acc,
             sem0, sem1, wsem0, wsem1):
        cid = lax.axis_index("c")
        sid = lax.axis_index("s")
        wid = cid * _NS + sid

        def zf(i, carry):
            zbuf[i // 8, pl.ds((i % 8) * 16, 16)] = jnp.zeros(
                (16,), jnp.float32)
            return carry

        lax.fori_loop(0, _WC * d // 16, zf, 0)
        for k in range(zr // _WC):
            pltpu.sync_copy(zbuf, acc.at[pl.ds(sid * zr + k * _WC, _WC)])
        if zr % _WC:
            pltpu.sync_copy(zbuf.at[pl.ds(0, zr % _WC)],
                            acc.at[pl.ds(sid * zr + zr - zr % _WC,
                                         zr % _WC)])

        pltpu.sync_copy(ei_hbm.at[:, pl.ds(wid * _NBT * _BL, bps * _BL)],
                        slab)
        plsc.subcore_barrier()

        def gather(ssb, buf, sem):
            pltpu.async_copy(y_hbm.at[ssb], buf, sem)

        def gwait(ssb, buf, sem):
            pltpu.make_async_copy(y_hbm.at[ssb], buf, sem).wait()

        for p in range(_NPH):
            if p > 0:
                pltpu.sync_copy(
                    ei_hbm.at[:, pl.ds((wid * _NBT + p * bps) * _BL,
                                       bps * _BL)],
                    slab)
            _stage(slab, 0, 0, ss0, nv)
            gather(ss0, buf0, sem0)

            def chunk2(i, carry):
                j0 = 2 * i
                j1 = j0 + 1
                j2 = j0 + 2
                _stage(slab, 0, j1 * _CHA, ss1, nv)
                gather(ss1, buf1, sem1)
                _stage(slab, 1, j0 * _CHA, ds0, nv)
                gwait(ss0, buf0, sem0)
                pltpu.sync_copy(buf0, acc.at[ds0], add=True)

                @pl.when(j2 < cps)
                def _():
                    _stage(slab, 0, j2 * _CHA, ss0, nv)
                    gather(ss0, buf0, sem0)

                _stage(slab, 1, j1 * _CHA, ds1, nv)
                gwait(ss1, buf1, sem1)
                pltpu.sync_copy(buf1, acc.at[ds1], add=True)
                return carry

            lax.fori_loop(0, cps // 2, chunk2, 0)

        @pl.when(wid < nxt)
        def _():
            pltpu.sync_copy(
                ei_hbm.at[:, pl.ds((_NBT * _NW + wid) * _BL, _BL)], xbuf)
            for q in range(_BL // _CHA):
                _stage(xbuf, 0, q * _CHA, ss0, nv)
                pltpu.async_copy(y_hbm.at[ss0], buf0, sem0).wait()
                _stage(xbuf, 1, q * _CHA, ds0, nv)
                pltpu.sync_copy(buf0, acc.at[ds0], add=True)

        plsc.subcore_barrier()

        def writeout(outref):
            bufs = [zbuf, buf0.at[pl.ds(0, _WC)]]
            rsem = [sem0, sem1]
            wsem = [wsem0, wsem1]

            def rslice(k):
                return acc.at[pl.ds(wsid * wr + k * _WC, _WC)]

            def oslice(k):
                return outref.at[pl.ds(wsid * wr + k * _WC, _WC)]

            pltpu.async_copy(rslice(0), bufs[0], rsem[0])
            for k in range(nwc):
                i = k % 2
                pltpu.make_async_copy(rslice(k), bufs[i], rsem[i]).wait()
                pltpu.async_copy(bufs[i], oslice(k), wsem[i])
                if k + 1 < nwc:
                    if k >= 1:
                        pltpu.make_async_copy(bufs[1 - i], oslice(k - 1),
                                              wsem[1 - i]).wait()
                    pltpu.async_copy(rslice(k + 1), bufs[1 - i], rsem[1 - i])
            for k in (nwc - 2, nwc - 1):
                pltpu.make_async_copy(bufs[k % 2], oslice(k),
                                      wsem[k % 2]).wait()

        wsid = sid - 4

        @pl.when(jnp.logical_and(jnp.logical_and(wsid >= 0, wsid < _WT),
                                 cid == 0))
        def _():
            writeout(out0)

        @pl.when(jnp.logical_and(jnp.logical_and(wsid >= 0, wsid < _WT),
                                 cid == 1))
        def _():
            writeout(out1)

    return pl.kernel(
        body,
        out_type=[jax.ShapeDtypeStruct((n, d), jnp.float32),
                  jax.ShapeDtypeStruct((n, d), jnp.float32)],
        mesh=_mesh,
        scratch_types=[
            pltpu.VMEM((2, _NBT // _NPH * _BL), jnp.int32),
            pltpu.VMEM((2, _BL), jnp.int32),
            pltpu.VMEM((_CHA, d), jnp.float32),
            pltpu.VMEM((_CHA, d), jnp.float32),
            pltpu.VMEM((_CHA,), jnp.int32),
            pltpu.VMEM((_CHA,), jnp.int32),
            pltpu.VMEM((_CHA,), jnp.int32),
            pltpu.VMEM((_CHA,), jnp.int32),
            pltpu.VMEM((_WC, d), jnp.float32),
            pltpu.VMEM_SHARED((n, d), jnp.float32),
            pltpu.SemaphoreType.DMA,
            pltpu.SemaphoreType.DMA,
            pltpu.SemaphoreType.DMA,
            pltpu.SemaphoreType.DMA,
        ],
    )


def _dv_body(d0_ref, d1_ref, dv_ref):
    deg = d0_ref[...] + d1_ref[...] + 1.0
    dv_ref[...] = lax.rsqrt(deg)[:, None]


def _y_body(x_ref, w_ref, dv_ref, y_ref):
    xw = jnp.dot(x_ref[...], w_ref[...], preferred_element_type=jnp.float32)
    y_ref[...] = xw * dv_ref[...]


def _head_body(a0_ref, a1_ref, y_ref, dv_ref, bc_ref, wr_ref, br_ref, o_ref):
    s = a0_ref[...] + a1_ref[...] + y_ref[...]
    h = jnp.maximum(s * dv_ref[...] + bc_ref[...], 0.0)
    o_ref[...] = jnp.dot(h, wr_ref[...],
                         preferred_element_type=jnp.float32) + br_ref[...]


def kernel(x, edge_index, W_conv, b_conv, W_reg, b_reg):
    n, d = x.shape
    e = edge_index.shape[1]
    blk = n // 5

    deg0, deg1 = _deg_kernel(n, e)(edge_index)

    dinv = pl.pallas_call(
        _dv_body,
        out_shape=jax.ShapeDtypeStruct((n, 1), jnp.float32),
    )(deg0, deg1)

    y = pl.pallas_call(
        _y_body,
        grid=(n // blk,),
        in_specs=[
            pl.BlockSpec((blk, d), lambda i: (i, 0)),
            pl.BlockSpec((d, d), lambda i: (0, 0)),
            pl.BlockSpec((blk, 1), lambda i: (i, 0)),
        ],
        out_specs=pl.BlockSpec((blk, d), lambda i: (i, 0)),
        out_shape=jax.ShapeDtypeStruct((n, d), jnp.float32),
    )(x, W_conv, dinv)

    acc0, acc1 = _agg_kernel(n, e, d)(edge_index, y)

    out = pl.pallas_call(
        _head_body,
        grid=(n // blk,),
        in_specs=[
            pl.BlockSpec((blk, d), lambda i: (i, 0)),
            pl.BlockSpec((blk, d), lambda i: (i, 0)),
            pl.BlockSpec((blk, d), lambda i: (i, 0)),
            pl.BlockSpec((blk, 1), lambda i: (i, 0)),
            pl.BlockSpec((d,), lambda i: (0,)),
            pl.BlockSpec((d, 1), lambda i: (0, 0)),
            pl.BlockSpec((1,), lambda i: (0,)),
        ],
        out_specs=pl.BlockSpec((blk, 1), lambda i: (i, 0)),
        out_shape=jax.ShapeDtypeStruct((n, 1), jnp.float32),
    )(acc0, acc1, y, dinv, b_conv, W_reg, b_reg)

    return out


# speedup vs baseline: 1.1983x; 1.0009x over previous
import jax
import jax.numpy as jnp
from jax import lax
from jax.experimental import pallas as pl
from jax.experimental.pallas import tpu as pltpu
from jax.experimental.pallas import tpu_sc as plsc

_NC = 2
_NS = 16
_NW = _NC * _NS
_BL = 128
_NBT = 78
_CHD = 128
_DGR = 13
_CHA = 128
_NPH = 3

_mesh = plsc.VectorSubcoreMesh(core_axis_name="c", subcore_axis_name="s")


def _stage(slab, row, off, dsb, nv):
    for k in range(nv):
        dsb[pl.ds(k * 16, 16)] = slab[row, pl.ds(off + k * 16, 16)]


def _deg_kernel(n, e):
    nxt = e // _BL - _NBT * _NW

    def body(ei_hbm, deg0, deg1, slab, xbuf, ones_v, stg, dsb, acc, sem):
        cid = lax.axis_index("c")
        sid = lax.axis_index("s")
        wid = cid * _NS + sid

        for k in range(_CHD // 16):
            ones_v[pl.ds(k * 16, 16)] = jnp.ones((16,), jnp.float32)

        @pl.when(sid == 0)
        def _():
            def zf(i, carry):
                stg[pl.ds(i * 16, 16)] = jnp.zeros((16,), jnp.float32)
                return carry

            lax.fori_loop(0, n // 16, zf, 0)
            pltpu.sync_copy(stg, acc)

        pltpu.sync_copy(ei_hbm.at[:, pl.ds(wid * _NBT * _BL, _NBT * _BL)],
                        slab)
        plsc.subcore_barrier()

        def group(g, carry):
            for k in range(_DGR):
                _stage(slab, 1, (g * _DGR + k) * _CHD, dsb.at[k], _CHD // 16)
                pltpu.async_copy(ones_v, acc.at[dsb.at[k]], sem, add=True)
            for k in range(_DGR):
                pltpu.make_async_copy(ones_v, acc.at[dsb.at[k]], sem).wait()
            return carry

        lax.fori_loop(0, _NBT * _BL // _CHD // _DGR, group, 0)

        @pl.when(wid < nxt)
        def _():
            pltpu.sync_copy(
                ei_hbm.at[:, pl.ds((_NBT * _NW + wid) * _BL, _BL)], xbuf)
            for q in range(_BL // _CHD):
                _stage(xbuf, 1, q * _CHD, dsb.at[0], _CHD // 16)
                pltpu.sync_copy(ones_v, acc.at[dsb.at[0]], add=True)

        plsc.subcore_barrier()

        @pl.when(jnp.logical_and(sid == 4, cid == 0))
        def _():
            pltpu.sync_copy(acc, stg)
            pltpu.sync_copy(stg, deg0)

        @pl.when(jnp.logical_and(sid == 4, cid == 1))
        def _():
            pltpu.sync_copy(acc, stg)
            pltpu.sync_copy(stg, deg1)

    return pl.kernel(
        body,
        out_type=[jax.ShapeDtypeStruct((n,), jnp.float32),
                  jax.ShapeDtypeStruct((n,), jnp.float32)],
        mesh=_mesh,
        scratch_types=[
            pltpu.VMEM((2, _NBT * _BL), jnp.int32),
            pltpu.VMEM((2, _BL), jnp.int32),
            pltpu.VMEM((_CHD,), jnp.float32),
            pltpu.VMEM((n,), jnp.float32),
            pltpu.VMEM((_DGR, _CHD), jnp.int32),
            pltpu.VMEM_SHARED((n,), jnp.float32),
            pltpu.SemaphoreType.DMA,
        ],
    )


_WT = 10
_WC = 40


def _agg_kernel(n, e, d):
    bps = _NBT // _NPH
    cps = bps * _BL // _CHA
    nxt = e // _BL - _NBT * _NW
    wr = n // _WT
    nwc = wr // _WC
    nv = _CHA // 16

    zr = n // _NS

    def body(ei_hbm, y_hbm, out0, out1,
             slab, xbuf, buf0, buf1, ss0, ss1, ds0, ds1, zbuf, acc,
             sem0, sem1, wsem0, wsem1):
        cid = lax.axis_index("c")
        sid = lax.axis_index("s")
        wid = cid * _NS + sid

        def zf(i, carry):
            zbuf[i // 8, pl.ds((i % 8) * 16, 16)] = jnp.zeros(
                (16,), jnp.float32)
            return carry

        lax.fori_loop(0, _WC * d // 16, zf, 0)
        for k in range(zr // _WC):
            pltpu.sync_copy(zbuf, acc.at[pl.ds(sid * zr + k * _WC, _WC)])
        if zr % _WC:
            pltpu.sync_copy(zbuf.at[pl.ds(0, zr % _WC)],
                            acc.at[pl.ds(sid * zr + zr - zr % _WC,
                                         zr % _WC)])

        pltpu.sync_copy(ei_hbm.at[:, pl.ds(wid * _NBT * _BL, bps * _BL)],
                        slab)
        plsc.subcore_barrier()

        def gather(ssb, buf, sem):
            pltpu.async_copy(y_hbm.at[ssb], buf, sem)

        def gwait(ssb, buf, sem):
            pltpu.make_async_copy(y_hbm.at[ssb], buf, sem).wait()

        for p in range(_NPH):
            if p > 0:
                pltpu.sync_copy(
                    ei_hbm.at[:, pl.ds((wid * _NBT + p * bps) * _BL,
                                       bps * _BL)],
                    slab)
            _stage(slab, 0, 0, ss0, nv)
            gather(ss0, buf0, sem0)

            def chunk2(i, carry):
                j0 = 2 * i
                j1 = j0 + 1
                j2 = j0 + 2

                @pl.when(i > 0)
                def _():
                    pltpu.make_async_copy(buf1, acc.at[ds1], wsem1).wait()

                _stage(slab, 0, j1 * _CHA, ss1, nv)
                gather(ss1, buf1, sem1)
                _stage(slab, 1, j0 * _CHA, ds0, nv)
                gwait(ss0, buf0, sem0)
                pltpu.async_copy(buf0, acc.at[ds0], wsem0, add=True)

                @pl.when(j2 < cps)
                def _():
                    pltpu.make_async_copy(buf0, acc.at[ds0], wsem0).wait()
                    _stage(slab, 0, j2 * _CHA, ss0, nv)
                    gather(ss0, buf0, sem0)

                _stage(slab, 1, j1 * _CHA, ds1, nv)
                gwait(ss1, buf1, sem1)
                pltpu.async_copy(buf1, acc.at[ds1], wsem1, add=True)
                return carry

            lax.fori_loop(0, cps // 2, chunk2, 0)
            pltpu.make_async_copy(buf0, acc.at[ds0], wsem0).wait()
            pltpu.make_async_copy(buf1, acc.at[ds1], wsem1).wait()

        @pl.when(wid < nxt)
        def _():
            pltpu.sync_copy(
                ei_hbm.at[:, pl.ds((_NBT * _NW + wid) * _BL, _BL)], xbuf)
            for q in range(_BL // _CHA):
                _stage(xbuf, 0, q * _CHA, ss0, nv)
                pltpu.async_copy(y_hbm.at[ss0], buf0, sem0).wait()
                _stage(xbuf, 1, q * _CHA, ds0, nv)
                pltpu.sync_copy(buf0, acc.at[ds0], add=True)

        plsc.subcore_barrier()

        def writeout(outref):
            bufs = [zbuf, buf0.at[pl.ds(0, _WC)]]
            rsem = [sem0, sem1]
            wsem = [wsem0, wsem1]

            def rslice(k):
                return acc.at[pl.ds(wsid * wr + k * _WC, _WC)]

            def oslice(k):
                return outref.at[pl.ds(wsid * wr + k * _WC, _WC)]

            pltpu.async_copy(rslice(0), bufs[0], rsem[0])
            for k in range(nwc):
                i = k % 2
                pltpu.make_async_copy(rslice(k), bufs[i], rsem[i]).wait()
                pltpu.async_copy(bufs[i], oslice(k), wsem[i])
                if k + 1 < nwc:
                    if k >= 1:
                        pltpu.make_async_copy(bufs[1 - i], oslice(k - 1),
                                              wsem[1 - i]).wait()
                    pltpu.async_copy(rslice(k + 1), bufs[1 - i], rsem[1 - i])
            for k in (nwc - 2, nwc - 1):
                pltpu.make_async_copy(bufs[k % 2], oslice(k),
                                      wsem[k % 2]).wait()

        wsid = sid - 4

        @pl.when(jnp.logical_and(jnp.logical_and(wsid >= 0, wsid < _WT),
                                 cid == 0))
        def _():
            writeout(out0)

        @pl.when(jnp.logical_and(jnp.logical_and(wsid >= 0, wsid < _WT),
                                 cid == 1))
        def _():
            writeout(out1)

    return pl.kernel(
        body,
        out_type=[jax.ShapeDtypeStruct((n, d), jnp.float32),
                  jax.ShapeDtypeStruct((n, d), jnp.float32)],
        mesh=_mesh,
        scratch_types=[
            pltpu.VMEM((2, _NBT // _NPH * _BL), jnp.int32),
            pltpu.VMEM((2, _BL), jnp.int32),
            pltpu.VMEM((_CHA, d), jnp.float32),
            pltpu.VMEM((_CHA, d), jnp.float32),
            pltpu.VMEM((_CHA,), jnp.int32),
            pltpu.VMEM((_CHA,), jnp.int32),
            pltpu.VMEM((_CHA,), jnp.int32),
            pltpu.VMEM((_CHA,), jnp.int32),
            pltpu.VMEM((_WC, d), jnp.float32),
            pltpu.VMEM_SHARED((n, d), jnp.float32),
            pltpu.SemaphoreType.DMA,
            pltpu.SemaphoreType.DMA,
            pltpu.SemaphoreType.DMA,
            pltpu.SemaphoreType.DMA,
        ],
    )


def _dv_body(d0_ref, d1_ref, dv_ref):
    deg = d0_ref[...] + d1_ref[...] + 1.0
    dv_ref[...] = lax.rsqrt(deg)[:, None]


def _y_body(x_ref, w_ref, dv_ref, y_ref):
    xw = jnp.dot(x_ref[...], w_ref[...], preferred_element_type=jnp.float32)
    y_ref[...] = xw * dv_ref[...]


def _head_body(a0_ref, a1_ref, y_ref, dv_ref, bc_ref, wr_ref, br_ref, o_ref):
    s = a0_ref[...] + a1_ref[...] + y_ref[...]
    h = jnp.maximum(s * dv_ref[...] + bc_ref[...], 0.0)
    o_ref[...] = jnp.dot(h, wr_ref[...],
                         preferred_element_type=jnp.float32) + br_ref[...]


def kernel(x, edge_index, W_conv, b_conv, W_reg, b_reg):
    n, d = x.shape
    e = edge_index.shape[1]
    blk = n // 5

    deg0, deg1 = _deg_kernel(n, e)(edge_index)

    dinv = pl.pallas_call(
        _dv_body,
        out_shape=jax.ShapeDtypeStruct((n, 1), jnp.float32),
    )(deg0, deg1)

    y = pl.pallas_call(
        _y_body,
        grid=(n // blk,),
        in_specs=[
            pl.BlockSpec((blk, d), lambda i: (i, 0)),
            pl.BlockSpec((d, d), lambda i: (0, 0)),
            pl.BlockSpec((blk, 1), lambda i: (i, 0)),
        ],
        out_specs=pl.BlockSpec((blk, d), lambda i: (i, 0)),
        out_shape=jax.ShapeDtypeStruct((n, d), jnp.float32),
    )(x, W_conv, dinv)

    acc0, acc1 = _agg_kernel(n, e, d)(edge_index, y)

    out = pl.pallas_call(
        _head_body,
        grid=(n // blk,),
        in_specs=[
            pl.BlockSpec((blk, d), lambda i: (i, 0)),
            pl.BlockSpec((blk, d), lambda i: (i, 0)),
            pl.BlockSpec((blk, d), lambda i: (i, 0)),
            pl.BlockSpec((blk, 1), lambda i: (i, 0)),
            pl.BlockSpec((d,), lambda i: (0,)),
            pl.BlockSpec((d, 1), lambda i: (0, 0)),
            pl.BlockSpec((1,), lambda i: (0,)),
        ],
        out_specs=pl.BlockSpec((blk, 1), lambda i: (i, 0)),
        out_shape=jax.ShapeDtypeStruct((n, 1), jnp.float32),
    )(acc0, acc1, y, dinv, b_conv, W_reg, b_reg)

    return out
